# baseline (speedup 1.0000x reference)
import functools

import jax
import jax.numpy as jnp
from jax import lax
from jax.experimental import pallas as pl
from jax.experimental.pallas import tpu as pltpu
from jax.experimental.pallas import tpu_sc as plsc

BATCH = 16384
EMBED_DIM = 32
NUM_CORES = 2
NUM_SUBCORES = 16
NUM_WORKERS = NUM_CORES * NUM_SUBCORES
ROWS_PER_WORKER = BATCH // NUM_WORKERS


def _sc_gather4(uidx, iidx, ug, ig, um, im):
    mesh = plsc.VectorSubcoreMesh(core_axis_name="c", subcore_axis_name="s")
    row_ty = jax.ShapeDtypeStruct((BATCH, EMBED_DIM), jnp.float32)

    @functools.partial(
        pl.kernel,
        out_type=(row_ty, row_ty, row_ty, row_ty),
        mesh=mesh,
        compiler_params=pltpu.CompilerParams(use_tc_tiling_on_sc=False),
        scratch_types=[
            pltpu.VMEM((ROWS_PER_WORKER,), jnp.int32),
            pltpu.VMEM((ROWS_PER_WORKER,), jnp.int32),
            pltpu.VMEM((ROWS_PER_WORKER, EMBED_DIM), jnp.float32),
            pltpu.VMEM((ROWS_PER_WORKER, EMBED_DIM), jnp.float32),
            pltpu.VMEM((ROWS_PER_WORKER, EMBED_DIM), jnp.float32),
            pltpu.VMEM((ROWS_PER_WORKER, EMBED_DIM), jnp.float32),
            pltpu.SemaphoreType.DMA,
            pltpu.SemaphoreType.DMA,
            pltpu.SemaphoreType.DMA,
            pltpu.SemaphoreType.DMA,
        ],
    )
    def k(uidx_hbm, iidx_hbm, ug_hbm, ig_hbm, um_hbm, im_hbm,
          oug, oig, oum, oim,
          uix_v, iix_v, b0, b1, b2, b3, s0, s1, s2, s3):
        wid = lax.axis_index("s") * NUM_CORES + lax.axis_index("c")
        base = wid * ROWS_PER_WORKER
        sl = pl.ds(base, ROWS_PER_WORKER)
        pltpu.sync_copy(uidx_hbm.at[sl], uix_v)
        pltpu.sync_copy(iidx_hbm.at[sl], iix_v)
        c0 = pltpu.async_copy(ug_hbm.at[uix_v], b0, s0)
        c1 = pltpu.async_copy(ig_hbm.at[iix_v], b1, s1)
        c2 = pltpu.async_copy(um_hbm.at[uix_v], b2, s2)
        c3 = pltpu.async_copy(im_hbm.at[iix_v], b3, s3)
        c0.wait()
        pltpu.sync_copy(b0, oug.at[sl])
        c1.wait()
        pltpu.sync_copy(b1, oig.at[sl])
        c2.wait()
        pltpu.sync_copy(b2, oum.at[sl])
        c3.wait()
        pltpu.sync_copy(b3, oim.at[sl])

    return k(uidx, iidx, ug, ig, um, im)


def _mlp_body(ug_r, ig_r, um_r, im_r, w0, b0, w1, b1, w2, b2, w3, b3,
              wo, bo, out_r):
    f32 = jnp.float32
    um = um_r[...]
    im = im_r[...]
    h = um @ w0[0:EMBED_DIM, :] + im @ w0[EMBED_DIM:2 * EMBED_DIM, :]
    h = jnp.maximum(h + b0[...], 0.0)
    h = jnp.maximum(jnp.dot(h, w1[...], preferred_element_type=f32) + b1[...], 0.0)
    h = jnp.maximum(jnp.dot(h, w2[...], preferred_element_type=f32) + b2[...], 0.0)
    h = jnp.maximum(jnp.dot(h, w3[...], preferred_element_type=f32) + b3[...], 0.0)
    g = ug_r[...] * ig_r[...]
    pred = (jnp.dot(g, wo[0:EMBED_DIM, :], preferred_element_type=f32)
            + jnp.dot(h, wo[EMBED_DIM:, :], preferred_element_type=f32)
            + bo[...])
    out_r[...] = pred


def _tc_mlp(ug_rows, ig_rows, um_rows, im_rows,
            W0, b0, W1, b1, W2, b2, W3, b3, Wo, bo, block_batch=2048):
    n_blocks = BATCH // block_batch
    row_spec = pl.BlockSpec((block_batch, EMBED_DIM), lambda i: (i, 0))

    def full2d(a):
        return pl.BlockSpec(a.shape, lambda i: (0, 0))

    b0r, b1r, b2r, b3r = (b.reshape(1, -1) for b in (b0, b1, b2, b3))
    bor = bo.reshape(1, 1)
    out = pl.pallas_call(
        _mlp_body,
        grid=(n_blocks,),
        in_specs=[row_spec, row_spec, row_spec, row_spec,
                  full2d(W0), full2d(b0r), full2d(W1), full2d(b1r),
                  full2d(W2), full2d(b2r), full2d(W3), full2d(b3r),
                  full2d(Wo), full2d(bor)],
        out_specs=pl.BlockSpec((block_batch, 1), lambda i: (i, 0)),
        out_shape=jax.ShapeDtypeStruct((BATCH, 1), jnp.float32),
    )(ug_rows, ig_rows, um_rows, im_rows,
      W0, b0r, W1, b1r, W2, b2r, W3, b3r, Wo, bor)
    return jnp.squeeze(out, axis=-1)


def kernel(user_indices, item_indices, ug, ig, um, im,
           W0, b0, W1, b1, W2, b2, W3, b3, Wo, bo):
    uidx = user_indices.astype(jnp.int32)
    iidx = item_indices.astype(jnp.int32)
    ug_rows, ig_rows, um_rows, im_rows = _sc_gather4(uidx, iidx, ug, ig, um, im)
    return _tc_mlp(ug_rows, ig_rows, um_rows, im_rows,
                   W0, b0, W1, b1, W2, b2, W3, b3, Wo, bo)
